# baseline (speedup 1.0000x reference)
import jax
import jax.numpy as jnp
from jax.experimental import pallas as pl
from jax.experimental.pallas import tpu as pltpu


def _v_kernel(x_ref, w_ref, b_ref, v_ref):
    wbar = jnp.mean(w_ref[...], axis=1, keepdims=True)
    bbar = jnp.mean(b_ref[...])
    v_ref[...] = (
        jnp.dot(x_ref[...], wbar, preferred_element_type=jnp.float32,
                precision=jax.lax.Precision.HIGHEST) + bbar
    )


def _mv_kernel(adj_ref, v_ref, out_ref):
    out_ref[...] = jnp.dot(
        adj_ref[...], v_ref[...], preferred_element_type=jnp.float32,
        precision=jax.lax.Precision.HIGHEST,
    )


def kernel(x, adj, W, b):
    n, d = x.shape
    c = W.shape[1]
    v = pl.pallas_call(
        _v_kernel,
        out_shape=jax.ShapeDtypeStruct((n, 1), jnp.float32),
    )(x, W, b.reshape(1, c))

    bn = 400
    out = pl.pallas_call(
        _mv_kernel,
        grid=(n // bn,),
        in_specs=[
            pl.BlockSpec((bn, n), lambda i: (i, 0)),
            pl.BlockSpec((n, 1), lambda i: (0, 0)),
        ],
        out_specs=pl.BlockSpec((bn, 1), lambda i: (i, 0)),
        out_shape=jax.ShapeDtypeStruct((n, 1), jnp.float32),
    )(adj, v)
    return out[:, 0]


# speedup vs baseline: 2.4074x; 2.4074x over previous
import jax
import jax.numpy as jnp
from jax.experimental import pallas as pl
from jax.experimental.pallas import tpu as pltpu


def _v_kernel(x_ref, w_ref, b_ref, v_ref):
    wbar = jnp.mean(w_ref[...], axis=1, keepdims=True)
    bbar = jnp.mean(b_ref[...])
    v_ref[...] = (
        jnp.dot(x_ref[...], wbar, preferred_element_type=jnp.float32,
                precision=jax.lax.Precision.HIGHEST) + bbar
    )


def _mv_kernel(adj_ref, vrow_ref, out_ref):
    out_ref[...] = jnp.sum(adj_ref[...] * vrow_ref[...], axis=1, keepdims=True)


def kernel(x, adj, W, b):
    n, d = x.shape
    c = W.shape[1]
    v = pl.pallas_call(
        _v_kernel,
        out_shape=jax.ShapeDtypeStruct((n, 1), jnp.float32),
    )(x, W, b.reshape(1, c))

    vrow = v.reshape(1, n)
    bn = 400
    out = pl.pallas_call(
        _mv_kernel,
        grid=(n // bn,),
        in_specs=[
            pl.BlockSpec((bn, n), lambda i: (i, 0)),
            pl.BlockSpec((1, n), lambda i: (0, 0)),
        ],
        out_specs=pl.BlockSpec((bn, 1), lambda i: (i, 0)),
        out_shape=jax.ShapeDtypeStruct((n, 1), jnp.float32),
    )(adj, vrow)
    return out[:, 0]


# speedup vs baseline: 2.6220x; 1.0891x over previous
import jax
import jax.numpy as jnp
from jax.experimental import pallas as pl
from jax.experimental.pallas import tpu as pltpu


def _fused_kernel(x_ref, wt_ref, b_ref, adj_ref, out_ref, v_scr):
    @pl.when(pl.program_id(0) == 0)
    def _():
        wbar_row = jnp.mean(wt_ref[...], axis=0, keepdims=True)
        bbar = jnp.mean(b_ref[...])
        vrow = jax.lax.dot_general(
            wbar_row, x_ref[...],
            (((1,), (1,)), ((), ())),
            preferred_element_type=jnp.float32,
            precision=jax.lax.Precision.HIGHEST,
        )
        v_scr[...] = vrow + bbar

    out_ref[...] = jnp.sum(adj_ref[...] * v_scr[...], axis=1, keepdims=True)


def kernel(x, adj, W, b):
    n, d = x.shape
    c = W.shape[1]
    bn = 200
    out = pl.pallas_call(
        _fused_kernel,
        grid=(n // bn,),
        in_specs=[
            pl.BlockSpec((n, d), lambda i: (0, 0)),
            pl.BlockSpec((c, d), lambda i: (0, 0)),
            pl.BlockSpec((1, c), lambda i: (0, 0)),
            pl.BlockSpec((bn, n), lambda i: (i, 0)),
        ],
        out_specs=pl.BlockSpec((bn, 1), lambda i: (i, 0)),
        out_shape=jax.ShapeDtypeStruct((n, 1), jnp.float32),
        scratch_shapes=[pltpu.VMEM((1, n), jnp.float32)],
    )(x, W.T, b.reshape(1, c), adj)
    return out[:, 0]
